# Initial kernel scaffold; baseline (speedup 1.0000x reference)
import jax
import jax.numpy as jnp
from jax.experimental import pallas as pl


def kernel(regression, classification, detections):
    raise NotImplementedError("write your pallas kernel here")


# speedup vs baseline: 12.0495x; 12.0495x over previous
import functools

import jax
import jax.numpy as jnp
from jax.experimental import pallas as pl
from jax.experimental.pallas import tpu as pltpu

_NMS_THRESHOLD = 0.4
_MAX_BOXES = 300
_NEG = -1e30
_N = 20000
_C = 80
_ROWS = 160
_LANES = 128
_NPAD = _ROWS * _LANES


def _nms_kernel(det_t_ref, det_ref, out_ref, scores_ref, areas_ref):
    y1 = det_t_ref[0, 0]
    x1 = det_t_ref[0, 1]
    y2 = det_t_ref[0, 2]
    x2 = det_t_ref[0, 3]
    areas_ref[:, :] = (y2 - y1) * (x2 - x1)

    m = det_t_ref[0, 4]
    for c in range(5, 4 + _C):
        m = jnp.maximum(m, det_t_ref[0, c])
    scores_ref[:, :] = m

    fi = (jax.lax.broadcasted_iota(jnp.int32, (_ROWS, _LANES), 0) * _LANES
          + jax.lax.broadcasted_iota(jnp.int32, (_ROWS, _LANES), 1)
          ).astype(jnp.float32)
    li = jax.lax.broadcasted_iota(jnp.int32, (1, 4 + _C), 1).astype(jnp.float32)

    def step(i, _):
        scores = scores_ref[:, :]
        sel_score = jnp.max(scores)
        idx_f = jnp.min(jnp.where(scores == sel_score, fi, 1e9))
        idx = idx_f.astype(jnp.int32)

        row = det_ref[0, pl.ds(idx, 1), :]
        s_y1 = jnp.sum(jnp.where(li == 0, row, 0.0))
        s_x1 = jnp.sum(jnp.where(li == 1, row, 0.0))
        s_y2 = jnp.sum(jnp.where(li == 2, row, 0.0))
        s_x2 = jnp.sum(jnp.where(li == 3, row, 0.0))
        s_area = (s_y2 - s_y1) * (s_x2 - s_x1)

        yy1 = jnp.maximum(y1, s_y1)
        xx1 = jnp.maximum(x1, s_x1)
        yy2 = jnp.minimum(y2, s_y2)
        xx2 = jnp.minimum(x2, s_x2)
        inter = jnp.maximum(yy2 - yy1, 0.0) * jnp.maximum(xx2 - xx1, 0.0)
        union = areas_ref[:, :] + s_area - inter
        iou = inter / jnp.maximum(union, 1e-9)
        kill = (iou > _NMS_THRESHOLD) | (fi == idx_f)
        scores_ref[:, :] = jnp.where(kill, _NEG, scores)

        valid = jnp.where(sel_score > _NEG * 0.5, 1.0, 0.0)

        cmax = jnp.max(jnp.where(li >= 4.0, row, -jnp.inf))
        lab = jnp.min(jnp.where((li >= 4.0) & (row == cmax), li, 1e9)) - 4.0

        out_row = jnp.concatenate(
            [row[:, :4], jnp.full((1, 1), lab, jnp.float32), row[:, 4:]],
            axis=1) * valid
        out_ref[0, pl.ds(i, 1), :] = out_row
        return 0

    jax.lax.fori_loop(0, _MAX_BOXES, step, 0)


@jax.jit
def kernel(regression, classification, detections):
    del regression, classification
    b, n, d = detections.shape
    det_t = jnp.transpose(detections, (0, 2, 1))
    det_t = jnp.pad(det_t, ((0, 0), (0, 0), (0, _NPAD - n)),
                    constant_values=_NEG)
    det_t = det_t.reshape(b, d, _ROWS, _LANES)

    return pl.pallas_call(
        _nms_kernel,
        grid=(b,),
        in_specs=[
            pl.BlockSpec((1, d, _ROWS, _LANES), lambda i: (i, 0, 0, 0)),
            pl.BlockSpec((1, n, d), lambda i: (i, 0, 0)),
        ],
        out_specs=pl.BlockSpec((1, _MAX_BOXES, 5 + _C), lambda i: (i, 0, 0)),
        out_shape=jax.ShapeDtypeStruct((b, _MAX_BOXES, 5 + _C), jnp.float32),
        scratch_shapes=[
            pltpu.VMEM((_ROWS, _LANES), jnp.float32),
            pltpu.VMEM((_ROWS, _LANES), jnp.float32),
        ],
    )(det_t, detections)


# speedup vs baseline: 13.2347x; 1.0984x over previous
import jax
import jax.numpy as jnp
from jax.experimental import pallas as pl
from jax.experimental.pallas import tpu as pltpu

_NMS_THRESHOLD = 0.4
_MAX_BOXES = 300
_NEG = -1e30
_N = 20000
_C = 80
_ROWS = 160
_LANES = 128
_NPAD = _ROWS * _LANES
_B = 2


def _nms_kernel(det_t_ref, det_ref, out_ref, scores_ref, areas_ref):
    planes = []
    for b in range(_B):
        y1 = det_t_ref[b, 0]
        x1 = det_t_ref[b, 1]
        y2 = det_t_ref[b, 2]
        x2 = det_t_ref[b, 3]
        areas_ref[b] = (y2 - y1) * (x2 - x1)
        m = det_t_ref[b, 4]
        for c in range(5, 4 + _C):
            m = jnp.maximum(m, det_t_ref[b, c])
        scores_ref[b] = m
        planes.append((y1, x1, y2, x2))

    fi = (jax.lax.broadcasted_iota(jnp.int32, (_ROWS, _LANES), 0) * _LANES
          + jax.lax.broadcasted_iota(jnp.int32, (_ROWS, _LANES), 1)
          ).astype(jnp.float32)
    li = jax.lax.broadcasted_iota(jnp.int32, (1, 4 + _C), 1).astype(jnp.float32)

    def one_batch(b, i):
        y1, x1, y2, x2 = planes[b]
        scores = scores_ref[b]
        sel_score = jnp.max(scores)
        idx_f = jnp.min(jnp.where(scores == sel_score, fi, 1e9))
        idx = idx_f.astype(jnp.int32)

        row = det_ref[b, pl.ds(idx, 1), :]
        s_y1 = jnp.sum(jnp.where(li == 0, row, 0.0))
        s_x1 = jnp.sum(jnp.where(li == 1, row, 0.0))
        s_y2 = jnp.sum(jnp.where(li == 2, row, 0.0))
        s_x2 = jnp.sum(jnp.where(li == 3, row, 0.0))
        s_area = (s_y2 - s_y1) * (s_x2 - s_x1)

        yy1 = jnp.maximum(y1, s_y1)
        xx1 = jnp.maximum(x1, s_x1)
        yy2 = jnp.minimum(y2, s_y2)
        xx2 = jnp.minimum(x2, s_x2)
        inter = jnp.maximum(yy2 - yy1, 0.0) * jnp.maximum(xx2 - xx1, 0.0)
        union = areas_ref[b] + s_area - inter
        iou = inter / jnp.maximum(union, 1e-9)
        kill = (iou > _NMS_THRESHOLD) | (fi == idx_f)
        scores_ref[b] = jnp.where(kill, _NEG, scores)

        valid = jnp.where(sel_score > _NEG * 0.5, 1.0, 0.0)

        cmax = jnp.max(jnp.where(li >= 4.0, row, -jnp.inf))
        lab = jnp.min(jnp.where((li >= 4.0) & (row == cmax), li, 1e9)) - 4.0

        out_row = jnp.concatenate(
            [row[:, :4], jnp.full((1, 1), lab, jnp.float32), row[:, 4:]],
            axis=1) * valid
        out_ref[b, pl.ds(i, 1), :] = out_row

    def step(i, _):
        for b in range(_B):
            one_batch(b, i)
        return 0

    jax.lax.fori_loop(0, _MAX_BOXES, step, 0)


@jax.jit
def kernel(regression, classification, detections):
    del regression, classification
    b, n, d = detections.shape
    det_t = jnp.transpose(detections, (0, 2, 1))
    det_t = jnp.pad(det_t, ((0, 0), (0, 0), (0, _NPAD - n)),
                    constant_values=_NEG)
    det_t = det_t.reshape(b, d, _ROWS, _LANES)

    return pl.pallas_call(
        _nms_kernel,
        grid=(1,),
        in_specs=[
            pl.BlockSpec((b, d, _ROWS, _LANES), lambda i: (0, 0, 0, 0)),
            pl.BlockSpec((b, n, d), lambda i: (0, 0, 0)),
        ],
        out_specs=pl.BlockSpec((b, _MAX_BOXES, 5 + _C), lambda i: (0, 0, 0)),
        out_shape=jax.ShapeDtypeStruct((b, _MAX_BOXES, 5 + _C), jnp.float32),
        scratch_shapes=[
            pltpu.VMEM((_B, _ROWS, _LANES), jnp.float32),
            pltpu.VMEM((_B, _ROWS, _LANES), jnp.float32),
        ],
    )(det_t, detections)
